# baseline (speedup 1.0000x reference)
import jax
import jax.numpy as jnp
from jax import lax
from jax.experimental import pallas as pl
from jax.experimental.pallas import tpu as pltpu
from jax.experimental.pallas import tpu_sc as plsc

N_FIELDS = 26
VOCAB = 100000
EMB = 32
BATCH = 16384

NC = 2
NS = 16
NW = NC * NS
B_PER_W = BATCH // NW
ROWS_PER_CHUNK = 4
CHUNK = ROWS_PER_CHUNK * N_FIELDS
NCHUNKS = B_PER_W // ROWS_PER_CHUNK
NBUF = 4
LANES = 16


def _tec_body(idx_hbm, tbl_hbm, out_hbm, idx_v, gbuf, acc, ld_sem, g_sems):
  wid = lax.axis_index("s") * NC + lax.axis_index("c")

  pltpu.async_copy(idx_hbm.at[wid], idx_v, ld_sem).wait()

  for b in range(NBUF):
    pltpu.async_copy(tbl_hbm.at[idx_v.at[b]], gbuf.at[pl.ds(b * CHUNK, CHUNK)],
                     g_sems.at[b])

  def _step(ch, c):
    slot = lax.rem(ch, NBUF)
    pltpu.make_async_copy(tbl_hbm.at[idx_v.at[ch]],
                          gbuf.at[pl.ds(slot * CHUNK, CHUNK)],
                          g_sems.at[slot]).wait()

    gbase = slot * CHUNK

    def _row(r, c2):
      src = gbase + r * N_FIELDS
      p0 = [gbuf[src + i, pl.ds(0, LANES)] for i in range(4)]
      p1 = [gbuf[src + i, pl.ds(LANES, LANES)] for i in range(4)]
      for i in range(4, N_FIELDS):
        p0[i % 4] += gbuf[src + i, pl.ds(0, LANES)]
        p1[i % 4] += gbuf[src + i, pl.ds(LANES, LANES)]
      row = ch * ROWS_PER_CHUNK + r
      acc[row, pl.ds(0, LANES)] = (p0[0] + p0[1]) + (p0[2] + p0[3])
      acc[row, pl.ds(LANES, LANES)] = (p1[0] + p1[1]) + (p1[2] + p1[3])
      return c2

    lax.fori_loop(0, ROWS_PER_CHUNK, _row, 0, unroll=True)

    nxt = ch + NBUF

    @pl.when(nxt < NCHUNKS)
    def _():
      pltpu.async_copy(tbl_hbm.at[idx_v.at[nxt]],
                       gbuf.at[pl.ds(slot * CHUNK, CHUNK)], g_sems.at[slot])

    return c

  lax.fori_loop(0, NCHUNKS, _step, 0, unroll=False)

  pltpu.async_copy(acc, out_hbm.at[pl.ds(wid * B_PER_W, B_PER_W)],
                   ld_sem).wait()


@jax.jit
def kernel(x, tables):
  tbl_flat = tables.reshape(N_FIELDS * VOCAB, EMB)

  flat_idx = x.astype(jnp.int32) + (jnp.arange(N_FIELDS, dtype=jnp.int32)
                                    * VOCAB)[None, :]
  idx = flat_idx.reshape(NW, NCHUNKS, CHUNK)

  mesh = plsc.VectorSubcoreMesh(core_axis_name="c", subcore_axis_name="s")
  f = pl.kernel(
      _tec_body,
      out_type=jax.ShapeDtypeStruct((BATCH, EMB), jnp.float32),
      mesh=mesh,
      compiler_params=pltpu.CompilerParams(use_tc_tiling_on_sc=False),
      scratch_types=[
          pltpu.VMEM((NCHUNKS, CHUNK), jnp.int32),
          pltpu.VMEM((NBUF * CHUNK, EMB), jnp.float32),
          pltpu.VMEM((B_PER_W, EMB), jnp.float32),
          pltpu.SemaphoreType.DMA,
          pltpu.SemaphoreType.DMA((NBUF,)),
      ],
  )
  return f(idx, tbl_flat)
